# Initial kernel scaffold; baseline (speedup 1.0000x reference)
import functools

import jax
import jax.numpy as jnp
from jax import lax
from jax.experimental import pallas as pl
from jax.experimental.pallas import tpu as pltpu
from jax.experimental.pallas import tpu_sc as plsc

B = 16384
D = 64
NNEG = 20
NV = NNEG + 1
L = 16
NC = 2
NS = 16
NW = NC * NS
BW = B // NW
CB = 32
NCHUNK = BW // CB
GJ = 6
GN = CB * NV // GJ


def _sc_body(upos_hbm, vidx_hbm, uw_hbm, vw_hbm, pos_hbm, neg_hbm,
             uidx_v, vidx_v, urows, vrows, posb, negb, sem):
    wid = lax.axis_index("s") * NC + lax.axis_index("c")

    def chunk_body(c, carry):
        gbase = wid * BW + c * CB
        grow = (wid * NCHUNK + c) * GJ

        pltpu.sync_copy(upos_hbm.at[pl.ds(gbase, CB)], uidx_v)
        pltpu.sync_copy(vidx_hbm.at[pl.ds(grow, GJ)], vidx_v)

        copies = [pltpu.async_copy(uw_hbm.at[uidx_v], urows, sem)]
        for j in range(GJ):
            copies.append(pltpu.async_copy(
                vw_hbm.at[vidx_v.at[j]],
                vrows.at[pl.ds(j * GN, GN)], sem))
        for cp in copies:
            cp.wait()

        def bbody(b, carry2):
            rb = b * NV
            u = [urows[b, pl.ds(16 * k, 16)] for k in range(4)]
            v = [vrows[rb, pl.ds(16 * k, 16)] for k in range(4)]
            acc = [vrows[rb + 1, pl.ds(16 * k, 16)] for k in range(4)]
            for n in range(2, NV):
                for k in range(4):
                    acc[k] = acc[k] + vrows[rb + n, pl.ds(16 * k, 16)]
            pos = u[0] * v[0] + u[1] * v[1] + u[2] * v[2] + u[3] * v[3]
            neg = u[0] * acc[0] + u[1] * acc[1] + u[2] * acc[2] + u[3] * acc[3]
            posb[b, :] = pos
            negb[b, :] = neg
            return carry2

        lax.fori_loop(0, CB, bbody, 0, unroll=False)

        pltpu.sync_copy(posb, pos_hbm.at[pl.ds(gbase, CB)])
        pltpu.sync_copy(negb, neg_hbm.at[pl.ds(gbase, CB)])
        return carry

    lax.fori_loop(0, NCHUNK, chunk_body, 0, unroll=False)


_sc_call = functools.partial(
    pl.kernel,
    out_type=(jax.ShapeDtypeStruct((B, L), jnp.float32),
              jax.ShapeDtypeStruct((B, L), jnp.float32)),
    mesh=plsc.VectorSubcoreMesh(core_axis_name="c", subcore_axis_name="s"),
    scratch_types=[
        pltpu.VMEM((CB,), jnp.int32),
        pltpu.VMEM((GJ, GN), jnp.int32),
        pltpu.VMEM((CB, D), jnp.float32),
        pltpu.VMEM((CB * NV, D), jnp.float32),
        pltpu.VMEM((CB, L), jnp.float32),
        pltpu.VMEM((CB, L), jnp.float32),
        pltpu.SemaphoreType.DMA,
    ],
)(_sc_body)


def _loss_body(pos_ref, neg_ref, out_ref):
    score = jnp.sum(pos_ref[...], axis=1)
    nscore = jnp.sum(neg_ref[...], axis=1)

    def logsig(x):
        return jnp.minimum(x, 0.0) - jnp.log1p(jnp.exp(-jnp.abs(x)))

    out_ref[0, 0] = -jnp.mean(logsig(score) + logsig(-nscore))


_loss_call = pl.pallas_call(
    _loss_body,
    out_shape=jax.ShapeDtypeStruct((1, 1), jnp.float32),
    out_specs=pl.BlockSpec(memory_space=pltpu.SMEM),
)


def kernel(u_pos, v_pos, v_neg, u_weight, v_weight):
    vidx = jnp.concatenate([v_pos[:, None], v_neg], axis=1).reshape(-1, GN)
    pos_part, neg_part = _sc_call(u_pos, vidx, u_weight, v_weight)
    out = _loss_call(pos_part, neg_part)
    return out[0, 0]


# speedup vs baseline: 5.1859x; 5.1859x over previous
import functools

import jax
import jax.numpy as jnp
from jax import lax
from jax.experimental import pallas as pl
from jax.experimental.pallas import tpu as pltpu
from jax.experimental.pallas import tpu_sc as plsc

B = 16384
D = 64
NNEG = 20
NV = NNEG + 1
L = 16
NC = 2
NS = 16
NW = NC * NS
BW = B // NW
CB = 32
NCHUNK = BW // CB
GJ = 6
GN = CB * NV // GJ


def _sc_body(upos_hbm, vidx_hbm, uw_hbm, vw_hbm, pos_hbm, neg_hbm,
             uidx_v, vidx_v, urows, vrows, posb, negb, sem):
    wid = lax.axis_index("s") * NC + lax.axis_index("c")

    def chunk_body(c, carry):
        gbase = wid * BW + c * CB

        pltpu.sync_copy(upos_hbm.at[pl.ds(gbase, CB)], uidx_v)
        pltpu.sync_copy(vidx_hbm.at[pl.ds(gbase * NV, CB * NV)], vidx_v)

        copies = [pltpu.async_copy(uw_hbm.at[uidx_v], urows, sem)]
        for j in range(GJ):
            copies.append(pltpu.async_copy(
                vw_hbm.at[vidx_v.at[pl.ds(j * GN, GN)]],
                vrows.at[pl.ds(j * GN, GN)], sem))
        for cp in copies:
            cp.wait()

        def bbody(b, carry2):
            rb = b * NV
            u = [urows[b, pl.ds(16 * k, 16)] for k in range(4)]
            v = [vrows[rb, pl.ds(16 * k, 16)] for k in range(4)]
            acc = [vrows[rb + 1, pl.ds(16 * k, 16)] for k in range(4)]
            for n in range(2, NV):
                for k in range(4):
                    acc[k] = acc[k] + vrows[rb + n, pl.ds(16 * k, 16)]
            pos = u[0] * v[0] + u[1] * v[1] + u[2] * v[2] + u[3] * v[3]
            neg = u[0] * acc[0] + u[1] * acc[1] + u[2] * acc[2] + u[3] * acc[3]
            posb[b, :] = pos
            negb[b, :] = neg
            return carry2

        lax.fori_loop(0, CB, bbody, 0, unroll=False)

        pltpu.sync_copy(posb, pos_hbm.at[pl.ds(gbase, CB)])
        pltpu.sync_copy(negb, neg_hbm.at[pl.ds(gbase, CB)])
        return carry

    lax.fori_loop(0, NCHUNK, chunk_body, 0, unroll=False)


_sc_call = functools.partial(
    pl.kernel,
    out_type=(jax.ShapeDtypeStruct((B, L), jnp.float32),
              jax.ShapeDtypeStruct((B, L), jnp.float32)),
    mesh=plsc.VectorSubcoreMesh(core_axis_name="c", subcore_axis_name="s"),
    compiler_params=pltpu.CompilerParams(use_tc_tiling_on_sc=False),
    scratch_types=[
        pltpu.VMEM((CB,), jnp.int32),
        pltpu.VMEM((CB * NV,), jnp.int32),
        pltpu.VMEM((CB, D), jnp.float32),
        pltpu.VMEM((CB * NV, D), jnp.float32),
        pltpu.VMEM((CB, L), jnp.float32),
        pltpu.VMEM((CB, L), jnp.float32),
        pltpu.SemaphoreType.DMA,
    ],
)(_sc_body)


def _loss_body(pos_ref, neg_ref, out_ref):
    score = jnp.sum(pos_ref[...], axis=1)
    nscore = jnp.sum(neg_ref[...], axis=1)

    def logsig(x):
        return jnp.minimum(x, 0.0) - jnp.log1p(jnp.exp(-jnp.abs(x)))

    out_ref[0, 0] = -jnp.mean(logsig(score) + logsig(-nscore))


_loss_call = pl.pallas_call(
    _loss_body,
    out_shape=jax.ShapeDtypeStruct((1, 1), jnp.float32),
    out_specs=pl.BlockSpec(memory_space=pltpu.SMEM),
)


def kernel(u_pos, v_pos, v_neg, u_weight, v_weight):
    vidx = jnp.concatenate([v_pos[:, None], v_neg], axis=1).reshape(-1)
    pos_part, neg_part = _sc_call(u_pos, vidx, u_weight, v_weight)
    out = _loss_call(pos_part, neg_part)
    return out[0, 0]
